# baseline (speedup 1.0000x reference)
import functools

import jax
import jax.numpy as jnp
from jax import lax
from jax.experimental import pallas as pl
from jax.experimental.pallas import tpu as pltpu
from jax.experimental.pallas import tpu_sc as plsc

_info = plsc.get_sparse_core_info()
_NC, _NS = _info.num_cores, _info.num_subcores
_NW = _NC * _NS

_NB = 8


def _gather_kernel(B, L, D, idx_hbm, table_hbm, out_hbm,
                   rows0, rows1, iv0, iv1, g0, g1, s0, s1, is0, is1):
    per_w = B // _NW
    n_ch = per_w // _NB
    cl = _NB * L
    wid = lax.axis_index("s") * _NC + lax.axis_index("c")
    base = wid * per_w

    rows = (rows0, rows1)
    ivec = (iv0, iv1)
    gsem = (g0, g1)
    ssem = (s0, s1)
    isem = (is0, is1)

    blocks = [(o, 0, 16) for o in range(0, cl - 15, 16)]
    if cl % 16:
        blocks.append((cl - 16, 16 - cl % 16, 16))

    def idx_start(c, p):
        pltpu.async_copy(
            idx_hbm.at[pl.ds(base * L + c * cl, cl)], ivec[p], isem[p])

    def idx_wait(p):
        pltpu.make_async_copy(
            idx_hbm.at[pl.ds(0, cl)], ivec[p], isem[p]).wait()

    def issue_chunk(c, p):
        for (o, j_lo, j_hi) in blocks:
            v = ivec[p][pl.ds(o, 16)]
            for j in range(j_lo, j_hi):
                i = v[j]
                r = o + j
                pltpu.async_copy(
                    table_hbm.at[pl.ds(i, 1)],
                    rows[p].at[r // L, pl.ds(r % L, 1)], gsem[p])

    def wait_gathers(p):
        pltpu.make_async_copy(
            out_hbm.at[pl.ds(0, _NB)], rows[p], gsem[p]).wait()

    def store_chunk(c, p):
        pltpu.async_copy(
            rows[p], out_hbm.at[pl.ds(base + c * _NB, _NB)], ssem[p])

    def wait_store(p):
        pltpu.make_async_copy(
            rows[p], out_hbm.at[pl.ds(0, _NB)], ssem[p]).wait()

    idx_start(0, 0)
    idx_start(1, 1)

    def body(c, p):
        @pl.when(c >= 2)
        def _():
            wait_gathers(p)
            store_chunk(c - 2, p)
            wait_store(p)

        idx_wait(p)
        issue_chunk(c, p)
        idx_start(jnp.minimum(c + 2, n_ch - 1), p)

    def pair(t, carry):
        body(2 * t, 0)
        body(2 * t + 1, 1)
        return carry

    lax.fori_loop(0, n_ch // 2, pair, 0)
    wait_gathers(0)
    store_chunk(n_ch - 2, 0)
    wait_gathers(1)
    store_chunk(n_ch - 1, 1)
    idx_wait(0)
    idx_wait(1)
    wait_store(0)
    wait_store(1)


def kernel(word_indexes, W):
    B, L = word_indexes.shape
    V, D = W.shape
    assert B % (_NW * _NB * 2) == 0

    idx = word_indexes.reshape(B * L).astype(jnp.int32)
    mesh = plsc.VectorSubcoreMesh(core_axis_name="c", subcore_axis_name="s")
    k = pl.kernel(
        functools.partial(_gather_kernel, B, L, D),
        mesh=mesh,
        out_type=jax.ShapeDtypeStruct((B, L, D), jnp.float32),
        scratch_types=[
            pltpu.VMEM((_NB, L, D), jnp.float32),
            pltpu.VMEM((_NB, L, D), jnp.float32),
            pltpu.VMEM((_NB * L,), jnp.int32),
            pltpu.VMEM((_NB * L,), jnp.int32),
            pltpu.SemaphoreType.DMA,
            pltpu.SemaphoreType.DMA,
            pltpu.SemaphoreType.DMA,
            pltpu.SemaphoreType.DMA,
            pltpu.SemaphoreType.DMA,
            pltpu.SemaphoreType.DMA,
        ],
        compiler_params=pltpu.CompilerParams(use_tc_tiling_on_sc=True),
    )
    return k(idx, W)


# speedup vs baseline: 1.0189x; 1.0189x over previous
import functools

import jax
import jax.numpy as jnp
from jax import lax
from jax.experimental import pallas as pl
from jax.experimental.pallas import tpu as pltpu
from jax.experimental.pallas import tpu_sc as plsc

_info = plsc.get_sparse_core_info()
_NC, _NS = _info.num_cores, _info.num_subcores
_NW = _NC * _NS

_NB = 4


def _gather_kernel(B, L, D, idx_hbm, table_hbm, out_hbm,
                   rows0, rows1, iv0, iv1, g0, g1, s0, s1, is0, is1):
    per_w = B // _NW
    n_ch = per_w // _NB
    cl = _NB * L
    wid = lax.axis_index("s") * _NC + lax.axis_index("c")
    base = wid * per_w

    rows = (rows0, rows1)
    ivec = (iv0, iv1)
    gsem = (g0, g1)
    ssem = (s0, s1)
    isem = (is0, is1)

    blocks = [(o, 0, 16) for o in range(0, cl - 15, 16)]
    if cl % 16:
        blocks.append((cl - 16, 16 - cl % 16, 16))

    def idx_start(c, p):
        pltpu.async_copy(
            idx_hbm.at[pl.ds(base * L + c * cl, cl)], ivec[p], isem[p])

    def idx_wait(p):
        pltpu.make_async_copy(
            idx_hbm.at[pl.ds(0, cl)], ivec[p], isem[p]).wait()

    def issue_chunk(c, p):
        for (o, j_lo, j_hi) in blocks:
            v = ivec[p][pl.ds(o, 16)]
            for j in range(j_lo, j_hi):
                i = v[j]
                r = o + j
                pltpu.async_copy(
                    table_hbm.at[pl.ds(i, 1)],
                    rows[p].at[r // L, pl.ds(r % L, 1)], gsem[p])

    def wait_gathers(p):
        pltpu.make_async_copy(
            out_hbm.at[pl.ds(0, _NB)], rows[p], gsem[p]).wait()

    def store_chunk(c, p):
        pltpu.async_copy(
            rows[p], out_hbm.at[pl.ds(base + c * _NB, _NB)], ssem[p])

    def wait_store(p):
        pltpu.make_async_copy(
            rows[p], out_hbm.at[pl.ds(0, _NB)], ssem[p]).wait()

    idx_start(0, 0)
    idx_start(1, 1)
    idx_wait(0)
    issue_chunk(0, 0)
    idx_start(2, 0)
    idx_wait(1)
    issue_chunk(1, 1)
    idx_start(3, 1)

    def body(c, p):
        wait_gathers(p)
        store_chunk(c - 2, p)
        wait_store(p)
        idx_wait(p)
        issue_chunk(c, p)
        idx_start(jnp.minimum(c + 2, n_ch - 1), p)

    def pair(t, carry):
        body(2 * t, 0)
        body(2 * t + 1, 1)
        return carry

    lax.fori_loop(1, n_ch // 2, pair, 0)
    wait_gathers(0)
    store_chunk(n_ch - 2, 0)
    wait_gathers(1)
    store_chunk(n_ch - 1, 1)
    idx_wait(0)
    idx_wait(1)
    wait_store(0)
    wait_store(1)


def kernel(word_indexes, W):
    B, L = word_indexes.shape
    V, D = W.shape
    assert B % (_NW * _NB * 2) == 0

    idx = word_indexes.reshape(B * L).astype(jnp.int32)
    mesh = plsc.VectorSubcoreMesh(core_axis_name="c", subcore_axis_name="s")
    k = pl.kernel(
        functools.partial(_gather_kernel, B, L, D),
        mesh=mesh,
        out_type=jax.ShapeDtypeStruct((B, L, D), jnp.float32),
        scratch_types=[
            pltpu.VMEM((_NB, L, D), jnp.float32),
            pltpu.VMEM((_NB, L, D), jnp.float32),
            pltpu.VMEM((_NB * L,), jnp.int32),
            pltpu.VMEM((_NB * L,), jnp.int32),
            pltpu.SemaphoreType.DMA,
            pltpu.SemaphoreType.DMA,
            pltpu.SemaphoreType.DMA,
            pltpu.SemaphoreType.DMA,
            pltpu.SemaphoreType.DMA,
            pltpu.SemaphoreType.DMA,
        ],
        compiler_params=pltpu.CompilerParams(use_tc_tiling_on_sc=True),
    )
    return k(idx, W)
